# baseline (speedup 1.0000x reference)
import functools

import jax
import jax.numpy as jnp
from jax import lax
from jax.experimental import pallas as pl
from jax.experimental.pallas import tpu as pltpu
from jax.experimental.pallas import tpu_sc as plsc

_C = 10
_T = 32


def _sc_body(buf_hbm, out_hbm, buf_v, out_v):
    wid = lax.axis_index("s") * 2 + lax.axis_index("c")

    @pl.when(wid == 0)
    def _():
        pltpu.sync_copy(buf_hbm, buf_v)
        v0 = buf_v[pl.ds(0, 16)]
        v1 = buf_v[pl.ds(16, 16)]
        v2 = buf_v[pl.ds(24, 16)]
        t = v2[_T - 24]
        w0 = v0[9] * v0[_C + 2]
        w1 = (v0[6] + v0[8]) * v0[_C + 1]
        w2 = (v0[3] + v0[5]) * v0[_C + 0]
        w3_nodiv = v0[_C + 3] + jnp.where(t <= v1[_C + 7 - 16], v0[_C + 4], 0.0)
        num = v0[_C + 5] * t
        den = v1[_C + 8 - 16]
        lane = lax.iota(jnp.int32, 16)
        lane3 = lane == 3
        base = jnp.where(
            lane == 0,
            w0,
            jnp.where(lane == 1, w1, jnp.where(lane == 2, w2, w3_nodiv)),
        )
        res = base + jnp.where(lane3, num, 0.0) / jnp.where(lane3, den, 1.0)
        out_v[...] = res
        pltpu.sync_copy(out_v, out_hbm)


_sc_call = functools.partial(
    pl.kernel,
    mesh=plsc.VectorSubcoreMesh(core_axis_name="c", subcore_axis_name="s"),
    out_type=jax.ShapeDtypeStruct((16,), jnp.float32),
    scratch_types=[
        pltpu.VMEM((40,), jnp.float32),
        pltpu.VMEM((16,), jnp.float32),
    ],
)(_sc_body)


@jax.jit
def kernel(y, w, c, t):
    buf = jnp.concatenate(
        [y, c, jnp.full((8,), t, jnp.float32)]
    )
    return _sc_call(buf)[:4]


# speedup vs baseline: 1.0334x; 1.0334x over previous
import functools

import jax
import jax.numpy as jnp
from jax import lax
from jax.experimental import pallas as pl
from jax.experimental.pallas import tpu as pltpu
from jax.experimental.pallas import tpu_sc as plsc


def _sc_body(y_hbm, c_hbm, t_hbm, out_hbm, y_v, c_v, t_v, out_v):
    @pl.when(lax.axis_index("s") == 0)
    def _():
        pltpu.sync_copy(y_hbm, y_v.at[pl.ds(0, 10)])
        pltpu.sync_copy(c_hbm, c_v.at[pl.ds(0, 22)])
        pltpu.sync_copy(t_hbm, t_v.at[pl.ds(0, 1)])
        yv = y_v[pl.ds(0, 16)]
        cv = c_v[pl.ds(0, 16)]
        t = t_v[pl.ds(0, 16)][0]
        w0 = yv[9] * cv[2]
        w1 = (yv[6] + yv[8]) * cv[1]
        w2 = (yv[3] + yv[5]) * cv[0]
        w3_nodiv = cv[3] + jnp.where(t <= cv[7], cv[4], 0.0)
        num = cv[5] * t
        den = cv[8]
        lane = lax.iota(jnp.int32, 16)
        lane3 = lane == 3
        base = jnp.where(
            lane == 0,
            w0,
            jnp.where(lane == 1, w1, jnp.where(lane == 2, w2, w3_nodiv)),
        )
        res = base + jnp.where(lane3, num, 0.0) / jnp.where(lane3, den, 1.0)
        out_v[...] = res
        pltpu.sync_copy(out_v.at[pl.ds(0, 4)], out_hbm)


_sc_call = functools.partial(
    pl.kernel,
    mesh=plsc.VectorSubcoreMesh(
        core_axis_name="c", subcore_axis_name="s", num_cores=1
    ),
    out_type=jax.ShapeDtypeStruct((4,), jnp.float32),
    scratch_types=[
        pltpu.VMEM((16,), jnp.float32),
        pltpu.VMEM((24,), jnp.float32),
        pltpu.VMEM((16,), jnp.float32),
        pltpu.VMEM((16,), jnp.float32),
    ],
)(_sc_body)


@jax.jit
def kernel(y, w, c, t):
    return _sc_call(y, c, t.reshape(1))
